# baseline (speedup 1.0000x reference)
import functools

import jax
import jax.numpy as jnp
from jax import lax
from jax.experimental import pallas as pl
from jax.experimental.pallas import tpu as pltpu
from jax.experimental.pallas import tpu_sc as plsc

B = 16384
D = 64
L = 16
NC, NS = 2, 16
NW = NC * NS
BPW = B // NW
CH = 128
NCH = BPW // CH
GPW = BPW // L

_mesh = plsc.VectorSubcoreMesh(core_axis_name="c", subcore_axis_name="s")


@functools.partial(
    pl.kernel,
    out_type=jax.ShapeDtypeStruct((B,), jnp.float32),
    mesh=_mesh,
    compiler_params=pltpu.CompilerParams(
        needs_layout_passes=False, use_tc_tiling_on_sc=False),
    scratch_types=[
        pltpu.VMEM((NCH, CH), jnp.int32),
        pltpu.VMEM((NCH, CH), jnp.int32),
        pltpu.VMEM((BPW, D), jnp.float32),
        pltpu.VMEM((BPW, D), jnp.float32),
        pltpu.VMEM((144,), jnp.float32),
        pltpu.VMEM((L * L,), jnp.float32),
        pltpu.VMEM((BPW,), jnp.float32),
        pltpu.SemaphoreType.DMA,
    ],
)
def _sc_forward(idx1_hbm, idx2_hbm, table_hbm, wb_hbm, out_hbm,
                idx1_v, idx2_v, rows1_v, rows2_v, wb_v, p_v, out_v, sem):
    wid = lax.axis_index("s") * NC + lax.axis_index("c")
    base = wid * BPW

    pltpu.sync_copy(wb_hbm, wb_v)
    for j in range(NCH):
        pltpu.sync_copy(idx1_hbm.at[pl.ds(base + j * CH, CH)], idx1_v.at[j])
        pltpu.sync_copy(idx2_hbm.at[pl.ds(base + j * CH, CH)], idx2_v.at[j])

    copies = []
    for j in range(NCH):
        copies.append(pltpu.async_copy(
            table_hbm.at[idx1_v.at[j]], rows1_v.at[pl.ds(j * CH, CH)], sem))
        copies.append(pltpu.async_copy(
            table_hbm.at[idx2_v.at[j]], rows2_v.at[pl.ds(j * CH, CH)], sem))
    for c in copies:
        c.wait()

    wvecs = [wb_v[pl.ds(j * L, L)] for j in range(2 * D // L)]
    bias = jnp.zeros((L,), jnp.float32) + wb_v[pl.ds(2 * D, L)][0]
    row_iota = lax.iota(jnp.int32, L)

    def group_body(g, carry):
        r0 = g * L
        for i in range(L):
            r = r0 + i
            p = rows1_v[r, pl.ds(0, L)] * wvecs[0]
            for j in range(1, D // L):
                p = p + rows1_v[r, pl.ds(j * L, L)] * wvecs[j]
            for j in range(D // L):
                p = p + rows2_v[r, pl.ds(j * L, L)] * wvecs[D // L + j]
            p_v[pl.ds(i * L, L)] = p
        acc = bias
        for i in range(L):
            acc = acc + plsc.load_gather(p_v, [row_iota * L + i])
        out_v[pl.ds(r0, L)] = acc
        return carry

    lax.fori_loop(0, GPW, group_body, 0)
    pltpu.sync_copy(out_v, out_hbm.at[pl.ds(base, BPW)])


def kernel(article1_idx, article2_idx, vector_tensor, W, b):
    wb = jnp.concatenate(
        [W.reshape(-1).astype(jnp.float32),
         jnp.pad(b.astype(jnp.float32), (0, 15))])
    out = _sc_forward(article1_idx.astype(jnp.int32),
                      article2_idx.astype(jnp.int32),
                      vector_tensor, wb)
    return out.reshape(B, 1)


# speedup vs baseline: 1.6379x; 1.6379x over previous
import functools

import jax
import jax.numpy as jnp
from jax import lax
from jax.experimental import pallas as pl
from jax.experimental.pallas import tpu as pltpu
from jax.experimental.pallas import tpu_sc as plsc

B = 16384
D = 64
L = 16
NC, NS = 2, 16
NW = NC * NS
BPW = B // NW
GPW = BPW // L

_mesh = plsc.VectorSubcoreMesh(core_axis_name="c", subcore_axis_name="s")


@functools.partial(
    pl.kernel,
    out_type=jax.ShapeDtypeStruct((B,), jnp.float32),
    mesh=_mesh,
    compiler_params=pltpu.CompilerParams(needs_layout_passes=False),
    scratch_types=[
        pltpu.VMEM((BPW,), jnp.int32),
        pltpu.VMEM((BPW,), jnp.int32),
        pltpu.VMEM((2 * L, D), jnp.float32),
        pltpu.VMEM((144,), jnp.float32),
        pltpu.VMEM((L * L,), jnp.float32),
        pltpu.VMEM((BPW,), jnp.float32),
        pltpu.SemaphoreType.DMA,
    ],
)
def _sc_forward(idx1_hbm, idx2_hbm, table_hbm, wb_hbm, out_hbm,
                idx1_v, idx2_v, rows_v, wb_v, p_v, out_v, sem):
    wid = lax.axis_index("s") * NC + lax.axis_index("c")
    base = wid * BPW

    pltpu.sync_copy(wb_hbm, wb_v)
    pltpu.sync_copy(idx1_hbm.at[pl.ds(base, BPW)], idx1_v)
    pltpu.sync_copy(idx2_hbm.at[pl.ds(base, BPW)], idx2_v)

    wvecs = [wb_v[pl.ds(j * L, L)] for j in range(2 * D // L)]
    bias = jnp.zeros((L,), jnp.float32) + wb_v[pl.ds(2 * D, L)][0]
    row_iota = lax.iota(jnp.int32, L)

    def group_body(g, carry):
        iv1 = idx1_v[pl.ds(g * L, L)]
        iv2 = idx2_v[pl.ds(g * L, L)]
        copies = []
        for i in range(L):
            copies.append(pltpu.async_copy(
                table_hbm.at[iv1[i]], rows_v.at[i], sem))
            copies.append(pltpu.async_copy(
                table_hbm.at[iv2[i]], rows_v.at[L + i], sem))
        for c in copies:
            c.wait()
        for i in range(L):
            p = rows_v[i, pl.ds(0, L)] * wvecs[0]
            for j in range(1, D // L):
                p = p + rows_v[i, pl.ds(j * L, L)] * wvecs[j]
            for j in range(D // L):
                p = p + rows_v[L + i, pl.ds(j * L, L)] * wvecs[D // L + j]
            p_v[pl.ds(i * L, L)] = p
        acc = bias
        for i in range(L):
            acc = acc + plsc.load_gather(p_v, [row_iota * L + i])
        out_v[pl.ds(g * L, L)] = acc
        return carry

    lax.fori_loop(0, GPW, group_body, 0)
    pltpu.sync_copy(out_v, out_hbm.at[pl.ds(base, BPW)])


def kernel(article1_idx, article2_idx, vector_tensor, W, b):
    wb = jnp.concatenate(
        [W.reshape(-1).astype(jnp.float32),
         jnp.pad(b.astype(jnp.float32), (0, 15))])
    out = _sc_forward(article1_idx.astype(jnp.int32),
                      article2_idx.astype(jnp.int32),
                      vector_tensor, wb)
    return out.reshape(B, 1)
